# baseline (speedup 1.0000x reference)
import jax
import jax.numpy as jnp
import numpy as np
from jax.experimental import pallas as pl
from jax.experimental.pallas import tpu as pltpu

_N = 10000
_F = 128
_H = 128
_C = 10
_K = 8
_ALPHA = 0.1

_BMQ = 400
_BM = 416
_NB = 25
_NPAD = _BM * _NB

_F8 = jnp.float8_e4m3fn
_SCALE = np.float32(2.0 ** -12)
_W09S = np.float32(0.9 * 2.0 ** -12)
_UNSCALE = np.float32(2.0 ** 96)


def _encoder_kernel(x_ref, w1_ref, b1_ref, w2_ref, b2_ref, z_ref):
    h = jax.lax.dot_general(
        x_ref[...], w1_ref[...], (((1,), (1,)), ((), ())),
        preferred_element_type=jnp.float32)
    h = jax.nn.relu(h + b1_ref[...])
    z = jax.lax.dot_general(
        h, w2_ref[...], (((1,), (1,)), ((), ())),
        preferred_element_type=jnp.float32)
    z_ref[...] = z + b2_ref[...]


def _hilo(c):
    hi = c.astype(_F8)
    lo = (c - hi.astype(jnp.float32)).astype(_F8)
    return hi, lo


def _quant_step0_kernel(adj_ref, z_ref, adj8_ref, hi1_ref, lo1_ref):
    i = pl.program_id(0)
    a8 = adj_ref[...].astype(_F8)
    adj8_ref[...] = a8
    zh, zl = _hilo(z_ref[...])
    y = (jax.lax.dot_general(a8, zh, (((1,), (0,)), ((), ())),
                             preferred_element_type=jnp.float32)
         + jax.lax.dot_general(a8, zl, (((1,), (0,)), ((), ())),
                               preferred_element_type=jnp.float32))
    zt = z_ref[pl.ds(i * _BMQ, _BMQ), :] * _SCALE
    c1 = y * _W09S + _ALPHA * zt
    hi1, lo1 = _hilo(c1)
    hi1_ref[...] = hi1
    lo1_ref[...] = lo1


def _prop_kernel(adj8_ref, z_ref, hi1_ref, lo1_ref, out_ref, hi_s, lo_s):
    k = pl.program_id(0)
    i = pl.program_id(1)

    @pl.when(jnp.logical_and(k == 0, i == 0))
    def _():
        hi_s[0, : _N, :] = hi1_ref[...]
        lo_s[0, : _N, :] = lo1_ref[...]

    b = jnp.remainder(k, 2)
    ph = hi_s[b, : _N, :]
    plo = lo_s[b, : _N, :]
    a8 = adj8_ref[...]
    y = (jax.lax.dot_general(a8, ph, (((1,), (0,)), ((), ())),
                             preferred_element_type=jnp.float32)
         + jax.lax.dot_general(a8, plo, (((1,), (0,)), ((), ())),
                               preferred_element_type=jnp.float32))
    ebits = jax.lax.shift_left(103 - 12 * k, 23)
    zscale = jax.lax.bitcast_convert_type(jnp.int32(ebits), jnp.float32)
    c = y * _W09S + _ALPHA * (z_ref[...] * zscale)
    hi, lo = _hilo(c)
    nb = jnp.remainder(k + 1, 2)
    hi_s[nb, pl.ds(i * _BM, _BM), :] = hi
    lo_s[nb, pl.ds(i * _BM, _BM), :] = lo

    @pl.when(k == _K - 2)
    def _():
        cur = c * _UNSCALE
        m = jnp.max(cur, axis=1, keepdims=True)
        shifted = cur - m
        lse = jnp.log(jnp.sum(jnp.exp(shifted), axis=1, keepdims=True))
        out_ref[pl.ds(i * _BM, _BM), :] = shifted - lse


def kernel(x, adj, W1, b1, W2, b2):
    z = pl.pallas_call(
        _encoder_kernel,
        grid=(_N // 1000,),
        in_specs=[
            pl.BlockSpec((1000, _F), lambda i: (i, 0)),
            pl.BlockSpec((_H, _F), lambda i: (0, 0)),
            pl.BlockSpec((1, _H), lambda i: (0, 0)),
            pl.BlockSpec((_C, _H), lambda i: (0, 0)),
            pl.BlockSpec((1, _C), lambda i: (0, 0)),
        ],
        out_specs=pl.BlockSpec((1000, _C), lambda i: (i, 0)),
        out_shape=jax.ShapeDtypeStruct((_N, _C), jnp.float32),
    )(x, W1, b1.reshape(1, _H), W2, b2.reshape(1, _C))

    adj8, hi1, lo1 = pl.pallas_call(
        _quant_step0_kernel,
        grid=(_N // _BMQ,),
        in_specs=[
            pl.BlockSpec((_BMQ, _N), lambda i: (i, 0)),
            pl.BlockSpec((_N, _C), lambda i: (0, 0)),
        ],
        out_specs=[
            pl.BlockSpec((_BMQ, _N), lambda i: (i, 0)),
            pl.BlockSpec((_BMQ, _C), lambda i: (i, 0)),
            pl.BlockSpec((_BMQ, _C), lambda i: (i, 0)),
        ],
        out_shape=[
            jax.ShapeDtypeStruct((_N, _N), _F8),
            jax.ShapeDtypeStruct((_N, _C), _F8),
            jax.ShapeDtypeStruct((_N, _C), _F8),
        ],
    )(adj, z)

    out = pl.pallas_call(
        _prop_kernel,
        grid=(_K - 1, _NB),
        in_specs=[
            pl.BlockSpec((_BM, _N), lambda k, i: (i, 0)),
            pl.BlockSpec((_BM, _C), lambda k, i: (i, 0)),
            pl.BlockSpec((_N, _C), lambda k, i: (0, 0)),
            pl.BlockSpec((_N, _C), lambda k, i: (0, 0)),
        ],
        out_specs=pl.BlockSpec((_NPAD, _C), lambda k, i: (0, 0)),
        out_shape=jax.ShapeDtypeStruct((_NPAD, _C), jnp.float32),
        scratch_shapes=[
            pltpu.VMEM((2, _NPAD, _C), _F8),
            pltpu.VMEM((2, _NPAD, _C), _F8),
        ],
    )(adj8, z, hi1, lo1)
    return out[:_N]


# speedup vs baseline: 1.0216x; 1.0216x over previous
import jax
import jax.numpy as jnp
from jax.experimental import pallas as pl
from jax.experimental.pallas import tpu as pltpu

_N = 10000
_F = 128
_H = 128
_C = 10
_K = 8
_ALPHA = 0.1

_BMQ = 400
_BM = 400


def _encoder_kernel(x_ref, w1_ref, b1_ref, w2_ref, b2_ref, z_ref):
    h = jax.lax.dot_general(
        x_ref[...], w1_ref[...], (((1,), (1,)), ((), ())),
        preferred_element_type=jnp.float32)
    h = jax.nn.relu(h + b1_ref[...])
    z = jax.lax.dot_general(
        h, w2_ref[...], (((1,), (1,)), ((), ())),
        preferred_element_type=jnp.float32)
    z_ref[...] = z + b2_ref[...]


def _quant_step0_kernel(adj_ref, z_ref, adj8_ref, cur1_ref):
    i = pl.program_id(0)
    a8 = adj_ref[...].astype(jnp.float8_e4m3fn)
    adj8_ref[...] = a8
    zb = z_ref[...].astype(jnp.bfloat16)
    y = jax.lax.dot_general(
        a8.astype(jnp.bfloat16), zb, (((1,), (0,)), ((), ())),
        preferred_element_type=jnp.float32)
    y = y * (1.0 - _ALPHA)
    y = y + _ALPHA * z_ref[pl.ds(i * _BMQ, _BMQ), :]
    cur1_ref[...] = y.astype(jnp.bfloat16)


def _prop_kernel(adj8_ref, z_ref, cur1_ref, out_ref, cur_ref):
    k = pl.program_id(0)
    i = pl.program_id(1)

    @pl.when(jnp.logical_and(k == 0, i == 0))
    def _():
        cur_ref[0] = cur1_ref[...]

    prev = cur_ref[jnp.remainder(k, 2)]
    y = jax.lax.dot_general(
        adj8_ref[...].astype(jnp.bfloat16), prev,
        (((1,), (0,)), ((), ())),
        preferred_element_type=jnp.float32)
    y = y * (1.0 - _ALPHA)
    y = y + _ALPHA * z_ref[...]
    cur_ref[jnp.remainder(k + 1, 2), pl.ds(i * _BM, _BM), :] = (
        y.astype(jnp.bfloat16))

    @pl.when(k == _K - 2)
    def _():
        m = jnp.max(y, axis=1, keepdims=True)
        shifted = y - m
        lse = jnp.log(jnp.sum(jnp.exp(shifted), axis=1, keepdims=True))
        out_ref[pl.ds(i * _BM, _BM), :] = shifted - lse


def kernel(x, adj, W1, b1, W2, b2):
    z = pl.pallas_call(
        _encoder_kernel,
        grid=(_N // 1000,),
        in_specs=[
            pl.BlockSpec((1000, _F), lambda i: (i, 0)),
            pl.BlockSpec((_H, _F), lambda i: (0, 0)),
            pl.BlockSpec((1, _H), lambda i: (0, 0)),
            pl.BlockSpec((_C, _H), lambda i: (0, 0)),
            pl.BlockSpec((1, _C), lambda i: (0, 0)),
        ],
        out_specs=pl.BlockSpec((1000, _C), lambda i: (i, 0)),
        out_shape=jax.ShapeDtypeStruct((_N, _C), jnp.float32),
    )(x, W1, b1.reshape(1, _H), W2, b2.reshape(1, _C))

    adj8, cur1 = pl.pallas_call(
        _quant_step0_kernel,
        grid=(_N // _BMQ,),
        in_specs=[
            pl.BlockSpec((_BMQ, _N), lambda i: (i, 0)),
            pl.BlockSpec((_N, _C), lambda i: (0, 0)),
        ],
        out_specs=[
            pl.BlockSpec((_BMQ, _N), lambda i: (i, 0)),
            pl.BlockSpec((_BMQ, _C), lambda i: (i, 0)),
        ],
        out_shape=[
            jax.ShapeDtypeStruct((_N, _N), jnp.float8_e4m3fn),
            jax.ShapeDtypeStruct((_N, _C), jnp.bfloat16),
        ],
    )(adj, z)

    out = pl.pallas_call(
        _prop_kernel,
        grid=(_K - 1, _N // _BM),
        in_specs=[
            pl.BlockSpec((_BM, _N), lambda k, i: (i, 0)),
            pl.BlockSpec((_BM, _C), lambda k, i: (i, 0)),
            pl.BlockSpec((_N, _C), lambda k, i: (0, 0)),
        ],
        out_specs=pl.BlockSpec((_N, _C), lambda k, i: (0, 0)),
        out_shape=jax.ShapeDtypeStruct((_N, _C), jnp.float32),
        scratch_shapes=[pltpu.VMEM((2, _N, _C), jnp.bfloat16)],
    )(adj8, z, cur1)
    return out
